# baseline (speedup 1.0000x reference)
import functools

import jax
import jax.numpy as jnp
from jax import lax
from jax.experimental import pallas as pl
from jax.experimental.pallas import tpu as pltpu
from jax.experimental.pallas import tpu_sc as plsc

NC = 2
NS = 16
L = 16

_MESH = plsc.VectorSubcoreMesh(core_axis_name="c", subcore_axis_name="s")
_SC_PARAMS = pltpu.CompilerParams(needs_layout_passes=False)


def _deg_body(n, e, dst_hbm, parts_hbm, dstbuf, deg):
    cid = lax.axis_index("c")
    sid = lax.axis_index("s")
    wid = sid * NC + cid
    epw = e // (NC * NS)
    pltpu.sync_copy(dst_hbm.at[pl.ds(wid * epw, epw)], dstbuf)
    zeros = jnp.zeros((L,), jnp.float32)

    @plsc.parallel_loop(0, n // L, unroll=8)
    def _(i):
        deg[pl.ds(i * L, L)] = zeros

    ones = jnp.ones((L,), jnp.float32)

    @plsc.parallel_loop(0, epw // L, unroll=8)
    def _(i):
        d16 = dstbuf[pl.ds(i * L, L)]
        plsc.addupdate_scatter(deg, [d16], ones)

    pltpu.sync_copy(deg, parts_hbm.at[wid])


def _deg_call(dst, n, e):
    body = functools.partial(_deg_body, n, e)
    return pl.kernel(
        body,
        out_type=jax.ShapeDtypeStruct((NC * NS, n), jnp.float32),
        mesh=_MESH,
        scratch_types=[
            pltpu.VMEM((e // (NC * NS),), jnp.int32),
            pltpu.VMEM((n,), jnp.float32),
        ],
        compiler_params=_SC_PARAMS,
    )(dst)


_CH = 10000


def _scatter_body(n, e, gt_hbm, pk_hbm, parts_hbm,
                  hcol, acc, pkbuf0, pkbuf1, sem0, sem1):
    cid = lax.axis_index("c")
    sid = lax.axis_index("s")
    eps = e // NC
    base = cid * eps
    nch = eps // _CH
    pkbufs = (pkbuf0, pkbuf1)
    sems = (sem0, sem1)

    def start(c, slot):
        off = base + lax.rem(c + sid, nch) * _CH
        cp = pltpu.make_async_copy(pk_hbm.at[pl.ds(off, _CH)],
                                   pkbufs[slot], sems[slot])
        cp.start()
        return cp

    pending = [None, None]
    pending[0] = start(0, 0)
    pltpu.sync_copy(gt_hbm.at[sid], hcol)
    zeros = jnp.zeros((L,), jnp.float32)

    @plsc.parallel_loop(0, n // L, unroll=8)
    def _(i):
        acc[pl.ds(i * L, L)] = zeros

    mask = jnp.full((L,), 0xFFFF, jnp.int32)

    for c in range(nch):
        slot = c % 2
        if c + 1 < nch:
            pending[(c + 1) % 2] = start(c + 1, (c + 1) % 2)
        pending[slot].wait()
        pb = pkbufs[slot]

        @plsc.parallel_loop(0, _CH // L, unroll=25)
        def _(i):
            p = pb[pl.ds(i * L, L)]
            s16 = lax.bitwise_and(p, mask)
            d16 = lax.shift_right_logical(p, 16)
            v = plsc.load_gather(hcol, [s16])
            plsc.addupdate_scatter(acc, [d16], v)

    pltpu.sync_copy(acc, parts_hbm.at[cid, sid])


def _scatter_call(gt, packed, n, e):
    body = functools.partial(_scatter_body, n, e)
    return pl.kernel(
        body,
        out_type=jax.ShapeDtypeStruct((NC, NS, n), jnp.float32),
        mesh=_MESH,
        scratch_types=[
            pltpu.VMEM((n,), jnp.float32),
            pltpu.VMEM((n,), jnp.float32),
            pltpu.VMEM((_CH,), jnp.int32),
            pltpu.VMEM((_CH,), jnp.int32),
            pltpu.SemaphoreType.DMA,
            pltpu.SemaphoreType.DMA,
        ],
        compiler_params=_SC_PARAMS,
    )(gt, packed)


_PREC = lax.Precision.HIGHEST


def _prep_body(x_ref, w1_ref, degp_ref, ei_ref, g_ref, dinv_ref, pk_ref):
    deg = jnp.sum(degp_ref[...], axis=0) + 1.0
    dinv = lax.rsqrt(deg)
    g = lax.dot_general(w1_ref[...], x_ref[...],
                        (((0,), (1,)), ((), ())),
                        precision=_PREC,
                        preferred_element_type=jnp.float32)
    g_ref[...] = g * dinv[None, :]
    dinv_ref[...] = dinv[None, :]
    pk_ref[...] = ei_ref[0:1, :] + (ei_ref[1:2, :] << 16)


def _prep_call(x, w1, deg_parts, edge_index, n, h, e):
    return pl.pallas_call(
        _prep_body,
        out_shape=[
            jax.ShapeDtypeStruct((h, n), jnp.float32),
            jax.ShapeDtypeStruct((1, n), jnp.float32),
            jax.ShapeDtypeStruct((1, e), jnp.int32),
        ],
    )(x, w1, deg_parts, edge_index)


def _mid_body(parts_ref, g_ref, dinv_ref, w2_ref, b1_ref, out_ref):
    tot = parts_ref[0] + parts_ref[1] + g_ref[...]
    dinv = dinv_ref[...]
    hid = jnp.maximum(tot * dinv + b1_ref[...], 0.0)
    g2 = lax.dot_general(w2_ref[...], hid,
                         (((0,), (0,)), ((), ())),
                         precision=_PREC,
                         preferred_element_type=jnp.float32)
    out_ref[...] = g2 * dinv


def _mid_call(parts, g, dinv, w2, b1, n, h):
    return pl.pallas_call(
        _mid_body,
        out_shape=jax.ShapeDtypeStruct((h, n), jnp.float32),
    )(parts, g, dinv, w2, b1.reshape(h, 1))


def _fin_body(parts_ref, g_ref, dinv_ref, wf_ref, b2_ref, bf_ref, out_ref):
    tot = parts_ref[0] + parts_ref[1] + g_ref[...]
    hid = jnp.maximum(tot * dinv_ref[...] + b2_ref[...], 0.0)
    o = lax.dot_general(wf_ref[...], hid,
                        (((0,), (0,)), ((), ())),
                        precision=_PREC,
                        preferred_element_type=jnp.float32)
    out_ref[...] = jax.nn.sigmoid(o + bf_ref[...])


def _fin_call(parts, g, dinv, wf, b2, bf, n, h):
    return pl.pallas_call(
        _fin_body,
        out_shape=jax.ShapeDtypeStruct((1, n), jnp.float32),
    )(parts, g, dinv, wf, b2.reshape(h, 1), bf.reshape(1, 1))


def kernel(x, edge_index, edge_attr, batch, W1, b1, W2, b2, Wf, bf):
    n = x.shape[0]
    e = edge_index.shape[1]
    h = W1.shape[1]
    dst = edge_index[1]

    deg_parts = _deg_call(dst, n, e)
    g1, dinv, packed = _prep_call(x, W1, deg_parts, edge_index, n, h, e)
    packed = packed.reshape(e)
    parts1 = _scatter_call(g1, packed, n, e)
    g2 = _mid_call(parts1, g1, dinv, W2, b1, n, h)
    parts2 = _scatter_call(g2, packed, n, e)
    out_t = _fin_call(parts2, g2, dinv, Wf, b2, bf, n, h)
    return out_t.reshape(n, 1)


# speedup vs baseline: 1.1337x; 1.1337x over previous
import functools

import jax
import jax.numpy as jnp
from jax import lax
from jax.experimental import pallas as pl
from jax.experimental.pallas import tpu as pltpu
from jax.experimental.pallas import tpu_sc as plsc

NC = 2
NS = 16
L = 16

_MESH = plsc.VectorSubcoreMesh(core_axis_name="c", subcore_axis_name="s")
_SC_PARAMS = pltpu.CompilerParams(needs_layout_passes=False)


def _deg_body(n, e, dst_hbm, parts_hbm, dstbuf, deg):
    cid = lax.axis_index("c")
    sid = lax.axis_index("s")
    wid = sid * NC + cid
    epw = e // (NC * NS)
    pltpu.sync_copy(dst_hbm.at[pl.ds(wid * epw, epw)], dstbuf)
    zeros = jnp.zeros((L,), jnp.float32)

    @plsc.parallel_loop(0, n // L, unroll=8)
    def _(i):
        deg[pl.ds(i * L, L)] = zeros

    ones = jnp.ones((L,), jnp.float32)

    @plsc.parallel_loop(0, epw // L, unroll=8)
    def _(i):
        d16 = dstbuf[pl.ds(i * L, L)]
        plsc.addupdate_scatter(deg, [d16], ones)

    pltpu.sync_copy(deg, parts_hbm.at[wid])


def _deg_call(dst, n, e):
    body = functools.partial(_deg_body, n, e)
    return pl.kernel(
        body,
        out_type=jax.ShapeDtypeStruct((NC * NS, n), jnp.float32),
        mesh=_MESH,
        scratch_types=[
            pltpu.VMEM((e // (NC * NS),), jnp.int32),
            pltpu.VMEM((n,), jnp.float32),
        ],
        compiler_params=_SC_PARAMS,
    )(dst)


_CH = 10000


def _scatter_body(n, e, gt_hbm, pk_hbm, parts_hbm,
                  hcol, acc, pkbuf0, pkbuf1, sem0, sem1):
    cid = lax.axis_index("c")
    sid = lax.axis_index("s")
    eps = e // NC
    base = cid * eps
    nch = eps // _CH
    pkbufs = (pkbuf0, pkbuf1)
    sems = (sem0, sem1)

    def start(c, slot):
        off = base + lax.rem(c + sid, nch) * _CH
        cp = pltpu.make_async_copy(pk_hbm.at[pl.ds(off, _CH)],
                                   pkbufs[slot], sems[slot])
        cp.start()
        return cp

    pending = [None, None]
    pending[0] = start(0, 0)
    pltpu.sync_copy(gt_hbm.at[sid], hcol)
    zeros = jnp.zeros((L,), jnp.float32)

    @plsc.parallel_loop(0, n // L, unroll=8)
    def _(i):
        acc[pl.ds(i * L, L)] = zeros

    mask = jnp.full((L,), 0xFFFF, jnp.int32)

    for c in range(nch):
        slot = c % 2
        if c + 1 < nch:
            pending[(c + 1) % 2] = start(c + 1, (c + 1) % 2)
        pending[slot].wait()
        pb = pkbufs[slot]

        @plsc.parallel_loop(0, _CH // L, unroll=8)
        def _(i):
            p = pb[pl.ds(i * L, L)]
            s16 = lax.bitwise_and(p, mask)
            d16 = lax.shift_right_logical(p, 16)
            v = plsc.load_gather(hcol, [s16])
            plsc.addupdate_scatter(acc, [d16], v)

    pltpu.sync_copy(acc, parts_hbm.at[cid, sid])


def _scatter_call(gt, packed, n, e):
    body = functools.partial(_scatter_body, n, e)
    return pl.kernel(
        body,
        out_type=jax.ShapeDtypeStruct((NC, NS, n), jnp.float32),
        mesh=_MESH,
        scratch_types=[
            pltpu.VMEM((n,), jnp.float32),
            pltpu.VMEM((n,), jnp.float32),
            pltpu.VMEM((_CH,), jnp.int32),
            pltpu.VMEM((_CH,), jnp.int32),
            pltpu.SemaphoreType.DMA,
            pltpu.SemaphoreType.DMA,
        ],
        compiler_params=_SC_PARAMS,
    )(gt, packed)


_PREC = lax.Precision.HIGHEST


def _prep_body(x_ref, w1_ref, degp_ref, src_ref, dst_ref, g_ref, dinv_ref, pk_ref):
    deg = jnp.sum(degp_ref[...], axis=0) + 1.0
    dinv = lax.rsqrt(deg)
    g = lax.dot_general(w1_ref[...], x_ref[...],
                        (((0,), (1,)), ((), ())),
                        precision=_PREC,
                        preferred_element_type=jnp.float32)
    g_ref[...] = g * dinv[None, :]
    dinv_ref[...] = dinv[None, :]
    pk_ref[...] = src_ref[...] + (dst_ref[...] << 16)


def _prep_call(x, w1, deg_parts, src2d, dst2d, n, h):
    return pl.pallas_call(
        _prep_body,
        out_shape=[
            jax.ShapeDtypeStruct((h, n), jnp.float32),
            jax.ShapeDtypeStruct((1, n), jnp.float32),
            jax.ShapeDtypeStruct(src2d.shape, jnp.int32),
        ],
    )(x, w1, deg_parts, src2d, dst2d)


def _mid_body(parts_ref, g_ref, dinv_ref, w2_ref, b1_ref, out_ref):
    tot = parts_ref[0] + parts_ref[1] + g_ref[...]
    dinv = dinv_ref[...]
    hid = jnp.maximum(tot * dinv + b1_ref[...], 0.0)
    g2 = lax.dot_general(w2_ref[...], hid,
                         (((0,), (0,)), ((), ())),
                         precision=_PREC,
                         preferred_element_type=jnp.float32)
    out_ref[...] = g2 * dinv


def _mid_call(parts, g, dinv, w2, b1, n, h):
    return pl.pallas_call(
        _mid_body,
        out_shape=jax.ShapeDtypeStruct((h, n), jnp.float32),
    )(parts, g, dinv, w2, b1.reshape(h, 1))


def _fin_body(parts_ref, g_ref, dinv_ref, wf_ref, b2_ref, bf_ref, out_ref):
    tot = parts_ref[0] + parts_ref[1] + g_ref[...]
    hid = jnp.maximum(tot * dinv_ref[...] + b2_ref[...], 0.0)
    o = lax.dot_general(wf_ref[...], hid,
                        (((0,), (0,)), ((), ())),
                        precision=_PREC,
                        preferred_element_type=jnp.float32)
    out_ref[...] = jax.nn.sigmoid(o + bf_ref[...])


def _fin_call(parts, g, dinv, wf, b2, bf, n, h):
    return pl.pallas_call(
        _fin_body,
        out_shape=jax.ShapeDtypeStruct((1, n), jnp.float32),
    )(parts, g, dinv, wf, b2.reshape(h, 1), bf.reshape(1, 1))


def kernel(x, edge_index, edge_attr, batch, W1, b1, W2, b2, Wf, bf):
    n = x.shape[0]
    e = edge_index.shape[1]
    h = W1.shape[1]
    dst = edge_index[1]

    deg_parts = _deg_call(dst, n, e)
    src2d = edge_index[0].reshape(e // 512, 512)
    dst2d = dst.reshape(e // 512, 512)
    g1, dinv, packed = _prep_call(x, W1, deg_parts, src2d, dst2d, n, h)
    packed = packed.reshape(e)
    parts1 = _scatter_call(g1, packed, n, e)
    g2 = _mid_call(parts1, g1, dinv, W2, b1, n, h)
    parts2 = _scatter_call(g2, packed, n, e)
    out_t = _fin_call(parts2, g2, dinv, Wf, b2, bf, n, h)
    return out_t.reshape(n, 1)


# speedup vs baseline: 1.1395x; 1.0051x over previous
import functools

import jax
import jax.numpy as jnp
from jax import lax
from jax.experimental import pallas as pl
from jax.experimental.pallas import tpu as pltpu
from jax.experimental.pallas import tpu_sc as plsc

NC = 2
NS = 16
L = 16

_MESH = plsc.VectorSubcoreMesh(core_axis_name="c", subcore_axis_name="s")
_SC_PARAMS = pltpu.CompilerParams(needs_layout_passes=False)


def _deg_body(n, e, dst_hbm, parts_hbm, dstbuf, deg):
    cid = lax.axis_index("c")
    sid = lax.axis_index("s")
    wid = sid * NC + cid
    epw = e // (NC * NS)
    pltpu.sync_copy(dst_hbm.at[pl.ds(wid * epw, epw)], dstbuf)
    zeros = jnp.zeros((L,), jnp.float32)

    @plsc.parallel_loop(0, n // L, unroll=8)
    def _(i):
        deg[pl.ds(i * L, L)] = zeros

    ones = jnp.ones((L,), jnp.float32)

    @plsc.parallel_loop(0, epw // L, unroll=8)
    def _(i):
        d16 = dstbuf[pl.ds(i * L, L)]
        plsc.addupdate_scatter(deg, [d16], ones)

    pltpu.sync_copy(deg, parts_hbm.at[wid])


def _deg_call(dst, n, e):
    body = functools.partial(_deg_body, n, e)
    return pl.kernel(
        body,
        out_type=jax.ShapeDtypeStruct((NC * NS, n), jnp.float32),
        mesh=_MESH,
        scratch_types=[
            pltpu.VMEM((e // (NC * NS),), jnp.int32),
            pltpu.VMEM((n,), jnp.float32),
        ],
        compiler_params=_SC_PARAMS,
    )(dst)


_CH = 20000


def _scatter_body(n, e, gt_hbm, pk_hbm, parts_hbm,
                  hcol, acc, pkbuf0, pkbuf1, sem0, sem1):
    cid = lax.axis_index("c")
    sid = lax.axis_index("s")
    eps = e // NC
    base = cid * eps
    nch = eps // _CH
    pkbufs = (pkbuf0, pkbuf1)
    sems = (sem0, sem1)

    def start(c, slot):
        off = base + lax.rem(c + sid, nch) * _CH
        cp = pltpu.make_async_copy(pk_hbm.at[pl.ds(off, _CH)],
                                   pkbufs[slot], sems[slot])
        cp.start()
        return cp

    pending = [None, None]
    pending[0] = start(0, 0)
    pltpu.sync_copy(gt_hbm.at[sid], hcol)
    zeros = jnp.zeros((L,), jnp.float32)

    @plsc.parallel_loop(0, n // L, unroll=8)
    def _(i):
        acc[pl.ds(i * L, L)] = zeros

    mask = jnp.full((L,), 0xFFFF, jnp.int32)

    for c in range(nch):
        slot = c % 2
        if c + 1 < nch:
            pending[(c + 1) % 2] = start(c + 1, (c + 1) % 2)
        pending[slot].wait()
        pb = pkbufs[slot]

        @plsc.parallel_loop(0, _CH // L, unroll=16)
        def _(i):
            p = pb[pl.ds(i * L, L)]
            s16 = lax.bitwise_and(p, mask)
            d16 = lax.shift_right_logical(p, 16)
            v = plsc.load_gather(hcol, [s16])
            plsc.addupdate_scatter(acc, [d16], v)

    pltpu.sync_copy(acc, parts_hbm.at[cid, sid])


def _scatter_call(gt, packed, n, e):
    body = functools.partial(_scatter_body, n, e)
    return pl.kernel(
        body,
        out_type=jax.ShapeDtypeStruct((NC, NS, n), jnp.float32),
        mesh=_MESH,
        scratch_types=[
            pltpu.VMEM((n,), jnp.float32),
            pltpu.VMEM((n,), jnp.float32),
            pltpu.VMEM((_CH,), jnp.int32),
            pltpu.VMEM((_CH,), jnp.int32),
            pltpu.SemaphoreType.DMA,
            pltpu.SemaphoreType.DMA,
        ],
        compiler_params=_SC_PARAMS,
    )(gt, packed)


_PREC = lax.Precision.HIGHEST


def _prep_body(x_ref, w1_ref, degp_ref, src_ref, dst_ref, g_ref, dinv_ref, pk_ref):
    deg = jnp.sum(degp_ref[...], axis=0) + 1.0
    dinv = lax.rsqrt(deg)
    g = lax.dot_general(w1_ref[...], x_ref[...],
                        (((0,), (1,)), ((), ())),
                        precision=_PREC,
                        preferred_element_type=jnp.float32)
    g_ref[...] = g * dinv[None, :]
    dinv_ref[...] = dinv[None, :]
    pk_ref[...] = src_ref[...] + (dst_ref[...] << 16)


def _prep_call(x, w1, deg_parts, src2d, dst2d, n, h):
    return pl.pallas_call(
        _prep_body,
        out_shape=[
            jax.ShapeDtypeStruct((h, n), jnp.float32),
            jax.ShapeDtypeStruct((1, n), jnp.float32),
            jax.ShapeDtypeStruct(src2d.shape, jnp.int32),
        ],
    )(x, w1, deg_parts, src2d, dst2d)


def _mid_body(parts_ref, g_ref, dinv_ref, w2_ref, b1_ref, out_ref):
    tot = parts_ref[0] + parts_ref[1] + g_ref[...]
    dinv = dinv_ref[...]
    hid = jnp.maximum(tot * dinv + b1_ref[...], 0.0)
    g2 = lax.dot_general(w2_ref[...], hid,
                         (((0,), (0,)), ((), ())),
                         precision=_PREC,
                         preferred_element_type=jnp.float32)
    out_ref[...] = g2 * dinv


def _mid_call(parts, g, dinv, w2, b1, n, h):
    return pl.pallas_call(
        _mid_body,
        out_shape=jax.ShapeDtypeStruct((h, n), jnp.float32),
    )(parts, g, dinv, w2, b1.reshape(h, 1))


def _fin_body(parts_ref, g_ref, dinv_ref, wf_ref, b2_ref, bf_ref, out_ref):
    tot = parts_ref[0] + parts_ref[1] + g_ref[...]
    hid = jnp.maximum(tot * dinv_ref[...] + b2_ref[...], 0.0)
    o = lax.dot_general(wf_ref[...], hid,
                        (((0,), (0,)), ((), ())),
                        precision=_PREC,
                        preferred_element_type=jnp.float32)
    out_ref[...] = jax.nn.sigmoid(o + bf_ref[...])


def _fin_call(parts, g, dinv, wf, b2, bf, n, h):
    return pl.pallas_call(
        _fin_body,
        out_shape=jax.ShapeDtypeStruct((1, n), jnp.float32),
    )(parts, g, dinv, wf, b2.reshape(h, 1), bf.reshape(1, 1))


def kernel(x, edge_index, edge_attr, batch, W1, b1, W2, b2, Wf, bf):
    n = x.shape[0]
    e = edge_index.shape[1]
    h = W1.shape[1]
    dst = edge_index[1]

    deg_parts = _deg_call(dst, n, e)
    src2d = edge_index[0].reshape(e // 512, 512)
    dst2d = dst.reshape(e // 512, 512)
    g1, dinv, packed = _prep_call(x, W1, deg_parts, src2d, dst2d, n, h)
    packed = packed.reshape(e)
    parts1 = _scatter_call(g1, packed, n, e)
    g2 = _mid_call(parts1, g1, dinv, W2, b1, n, h)
    parts2 = _scatter_call(g2, packed, n, e)
    out_t = _fin_call(parts2, g2, dinv, Wf, b2, bf, n, h)
    return out_t.reshape(n, 1)


# speedup vs baseline: 1.2947x; 1.1362x over previous
import functools

import jax
import jax.numpy as jnp
from jax import lax
from jax.experimental import pallas as pl
from jax.experimental.pallas import tpu as pltpu
from jax.experimental.pallas import tpu_sc as plsc

NC = 2
NS = 16
L = 16

_MESH = plsc.VectorSubcoreMesh(core_axis_name="c", subcore_axis_name="s")
_SC_PARAMS = pltpu.CompilerParams(needs_layout_passes=False)


def _deg_body(n, e, pk_hbm, parts_hbm, pkbuf, deg):
    cid = lax.axis_index("c")
    sid = lax.axis_index("s")
    wid = sid * NC + cid
    epw = e // (NC * NS)
    pltpu.sync_copy(pk_hbm.at[pl.ds(wid * epw, epw)], pkbuf)
    zeros = jnp.zeros((L,), jnp.float32)

    @plsc.parallel_loop(0, n // L, unroll=8)
    def _(i):
        deg[pl.ds(i * L, L)] = zeros

    ones = jnp.ones((L,), jnp.float32)

    @plsc.parallel_loop(0, epw // L, unroll=8)
    def _(i):
        d16 = lax.shift_right_logical(pkbuf[pl.ds(i * L, L)], 16)
        plsc.addupdate_scatter(deg, [d16], ones)

    pltpu.sync_copy(deg, parts_hbm.at[wid])


def _deg_call(packed, n, e):
    body = functools.partial(_deg_body, n, e)
    return pl.kernel(
        body,
        out_type=jax.ShapeDtypeStruct((NC * NS, n), jnp.float32),
        mesh=_MESH,
        scratch_types=[
            pltpu.VMEM((e // (NC * NS),), jnp.int32),
            pltpu.VMEM((n,), jnp.float32),
        ],
        compiler_params=_SC_PARAMS,
    )(packed)


_CH = 20000


def _scatter_body(n, e, gt_hbm, pk_hbm, parts_hbm,
                  hcol, acc, pkbuf0, pkbuf1, sem0, sem1):
    cid = lax.axis_index("c")
    sid = lax.axis_index("s")
    eps = e // NC
    base = cid * eps
    nch = eps // _CH
    pkbufs = (pkbuf0, pkbuf1)
    sems = (sem0, sem1)

    def start(c, slot):
        off = base + lax.rem(c + sid, nch) * _CH
        cp = pltpu.make_async_copy(pk_hbm.at[pl.ds(off, _CH)],
                                   pkbufs[slot], sems[slot])
        cp.start()
        return cp

    pending = [None, None]
    pending[0] = start(0, 0)
    pltpu.sync_copy(gt_hbm.at[sid], hcol)
    zeros = jnp.zeros((L,), jnp.float32)

    @plsc.parallel_loop(0, n // L, unroll=8)
    def _(i):
        acc[pl.ds(i * L, L)] = zeros

    mask = jnp.full((L,), 0xFFFF, jnp.int32)

    for c in range(nch):
        slot = c % 2
        if c + 1 < nch:
            pending[(c + 1) % 2] = start(c + 1, (c + 1) % 2)
        pending[slot].wait()
        pb = pkbufs[slot]

        @plsc.parallel_loop(0, _CH // L, unroll=16)
        def _(i):
            p = pb[pl.ds(i * L, L)]
            s16 = lax.bitwise_and(p, mask)
            d16 = lax.shift_right_logical(p, 16)
            v = plsc.load_gather(hcol, [s16])
            plsc.addupdate_scatter(acc, [d16], v)

    pltpu.sync_copy(acc, parts_hbm.at[cid, sid])


def _scatter_call(gt, packed, n, e):
    body = functools.partial(_scatter_body, n, e)
    return pl.kernel(
        body,
        out_type=jax.ShapeDtypeStruct((NC, NS, n), jnp.float32),
        mesh=_MESH,
        scratch_types=[
            pltpu.VMEM((n,), jnp.float32),
            pltpu.VMEM((n,), jnp.float32),
            pltpu.VMEM((_CH,), jnp.int32),
            pltpu.VMEM((_CH,), jnp.int32),
            pltpu.SemaphoreType.DMA,
            pltpu.SemaphoreType.DMA,
        ],
        compiler_params=_SC_PARAMS,
    )(gt, packed)


_PREC = lax.Precision.HIGHEST


def _pack_mm_body(ei_ref, x_ref, w1_ref, pk_ref, graw_ref):
    pk_ref[...] = ei_ref[0] + (ei_ref[1] << 16)
    graw_ref[...] = lax.dot_general(w1_ref[...], x_ref[...],
                                    (((0,), (1,)), ((), ())),
                                    precision=_PREC,
                                    preferred_element_type=jnp.float32)


def _pack_mm_call(edge_index, x, w1, n, h, e):
    return pl.pallas_call(
        _pack_mm_body,
        out_shape=[
            jax.ShapeDtypeStruct((e,), jnp.int32),
            jax.ShapeDtypeStruct((h, n), jnp.float32),
        ],
    )(edge_index, x, w1)


def _scale_body(degp_ref, graw_ref, g_ref, dinv_ref):
    deg = jnp.sum(degp_ref[...], axis=0) + 1.0
    dinv = lax.rsqrt(deg)
    g_ref[...] = graw_ref[...] * dinv[None, :]
    dinv_ref[...] = dinv[None, :]


def _scale_call(deg_parts, graw, n, h):
    return pl.pallas_call(
        _scale_body,
        out_shape=[
            jax.ShapeDtypeStruct((h, n), jnp.float32),
            jax.ShapeDtypeStruct((1, n), jnp.float32),
        ],
    )(deg_parts, graw)


def _mid_body(parts_ref, g_ref, dinv_ref, w2_ref, b1_ref, out_ref):
    tot = parts_ref[0] + parts_ref[1] + g_ref[...]
    dinv = dinv_ref[...]
    hid = jnp.maximum(tot * dinv + b1_ref[...], 0.0)
    g2 = lax.dot_general(w2_ref[...], hid,
                         (((0,), (0,)), ((), ())),
                         precision=_PREC,
                         preferred_element_type=jnp.float32)
    out_ref[...] = g2 * dinv


def _mid_call(parts, g, dinv, w2, b1, n, h):
    return pl.pallas_call(
        _mid_body,
        out_shape=jax.ShapeDtypeStruct((h, n), jnp.float32),
    )(parts, g, dinv, w2, b1.reshape(h, 1))


def _fin_body(parts_ref, g_ref, dinv_ref, wf_ref, b2_ref, bf_ref, out_ref):
    tot = parts_ref[0] + parts_ref[1] + g_ref[...]
    hid = jnp.maximum(tot * dinv_ref[...] + b2_ref[...], 0.0)
    o = lax.dot_general(wf_ref[...], hid,
                        (((0,), (0,)), ((), ())),
                        precision=_PREC,
                        preferred_element_type=jnp.float32)
    out_ref[...] = jax.nn.sigmoid(o + bf_ref[...])


def _fin_call(parts, g, dinv, wf, b2, bf, n, h):
    return pl.pallas_call(
        _fin_body,
        out_shape=jax.ShapeDtypeStruct((1, n), jnp.float32),
    )(parts, g, dinv, wf, b2.reshape(h, 1), bf.reshape(1, 1))


def kernel(x, edge_index, edge_attr, batch, W1, b1, W2, b2, Wf, bf):
    n = x.shape[0]
    e = edge_index.shape[1]
    h = W1.shape[1]
    packed, g1raw = _pack_mm_call(edge_index, x, W1, n, h, e)
    deg_parts = _deg_call(packed, n, e)
    g1, dinv = _scale_call(deg_parts, g1raw, n, h)
    parts1 = _scatter_call(g1, packed, n, e)
    g2 = _mid_call(parts1, g1, dinv, W2, b1, n, h)
    parts2 = _scatter_call(g2, packed, n, e)
    out_t = _fin_call(parts2, g2, dinv, Wf, b2, bf, n, h)
    return out_t.reshape(n, 1)


# speedup vs baseline: 1.3174x; 1.0175x over previous
import functools

import jax
import jax.numpy as jnp
from jax import lax
from jax.experimental import pallas as pl
from jax.experimental.pallas import tpu as pltpu
from jax.experimental.pallas import tpu_sc as plsc

NC = 2
NS = 16
L = 16

_MESH = plsc.VectorSubcoreMesh(core_axis_name="c", subcore_axis_name="s")
_SC_PARAMS = pltpu.CompilerParams(needs_layout_passes=False)


def _rsqrt16(x):
    i = plsc.bitcast(x, jnp.int32)
    i = jnp.int32(0x5F3759DF) - lax.shift_right_logical(i, 1)
    y = plsc.bitcast(i, jnp.float32)
    for _ in range(3):
        y = y * (1.5 - 0.5 * x * y * y)
    return y


_CH = 20000
_CW = 640


def _fused_body(n, e, gt_hbm, pk_hbm, parts_hbm, dinv_hbm,
                hcol, acc, pkbuf0, pkbuf1, degl, cbuf, dslice, dinvv,
                shared_deg, shared_dinv, sem0, sem1, gsem):
    cid = lax.axis_index("c")
    sid = lax.axis_index("s")
    n2 = NS * _CW
    eps = e // NC
    base = cid * eps
    nch = eps // _CH
    pkbufs = (pkbuf0, pkbuf1)
    sems = (sem0, sem1)
    epw = e // NS

    ca = pltpu.make_async_copy(pk_hbm.at[pl.ds(sid * epw, epw)], pkbuf0, sem0)
    ca.start()
    cg = pltpu.make_async_copy(gt_hbm.at[sid], hcol, gsem)
    cg.start()
    zeros = jnp.zeros((L,), jnp.float32)

    @plsc.parallel_loop(0, n2 // L, unroll=8)
    def _(i):
        degl[pl.ds(i * L, L)] = zeros

    @plsc.parallel_loop(0, n // L, unroll=8)
    def _(i):
        acc[pl.ds(i * L, L)] = zeros

    ca.wait()
    ones = jnp.ones((L,), jnp.float32)

    @plsc.parallel_loop(0, epw // L, unroll=8)
    def _(i):
        d16 = lax.shift_right_logical(pkbuf0[pl.ds(i * L, L)], 16)
        plsc.addupdate_scatter(degl, [d16], ones)

    pltpu.sync_copy(degl, shared_deg.at[sid])
    plsc.subcore_barrier()

    o = sid * _CW
    pltpu.sync_copy(shared_deg.at[:, pl.ds(o, _CW)], cbuf)

    @plsc.parallel_loop(0, _CW // L, unroll=4)
    def _(j):
        s = cbuf[0, pl.ds(j * L, L)]
        for k in range(1, NS):
            s = s + cbuf[k, pl.ds(j * L, L)]
        dslice[pl.ds(j * L, L)] = _rsqrt16(s + 1.0)

    pltpu.sync_copy(dslice, shared_dinv.at[pl.ds(o, _CW)])

    @pl.when(cid == 0)
    def _():
        pltpu.sync_copy(dslice, dinv_hbm.at[0, pl.ds(o, _CW)])

    plsc.subcore_barrier()

    cg.wait()
    pltpu.sync_copy(shared_dinv, dinvv)

    def start(c, slot):
        off = base + lax.rem(c + sid, nch) * _CH
        cp = pltpu.make_async_copy(pk_hbm.at[pl.ds(off, _CH)],
                                   pkbufs[slot], sems[slot])
        cp.start()
        return cp

    pending = [None, None]
    pending[0] = start(0, 0)

    @plsc.parallel_loop(0, n // L, unroll=8)
    def _(i):
        hcol[pl.ds(i * L, L)] = hcol[pl.ds(i * L, L)] * dinvv[pl.ds(i * L, L)]

    for c in range(nch):
        slot = c % 2
        if c + 1 < nch:
            pending[(c + 1) % 2] = start(c + 1, (c + 1) % 2)
        pending[slot].wait()
        pb = pkbufs[slot]
        mask = jnp.full((L,), 0xFFFF, jnp.int32)

        @plsc.parallel_loop(0, _CH // L, unroll=8)
        def _(i):
            p = pb[pl.ds(i * L, L)]
            s16 = lax.bitwise_and(p, mask)
            d16 = lax.shift_right_logical(p, 16)
            v = plsc.load_gather(hcol, [s16])
            plsc.addupdate_scatter(acc, [d16], v)

    pltpu.sync_copy(acc, parts_hbm.at[cid, sid])


def _fused_call(gtraw, packed, n, e):
    body = functools.partial(_fused_body, n, e)
    n2 = NS * _CW
    return pl.kernel(
        body,
        out_type=[
            jax.ShapeDtypeStruct((NC, NS, n), jnp.float32),
            jax.ShapeDtypeStruct((1, n2), jnp.float32),
        ],
        mesh=_MESH,
        scratch_types=[
            pltpu.VMEM((n,), jnp.float32),
            pltpu.VMEM((n,), jnp.float32),
            pltpu.VMEM((_CH,), jnp.int32),
            pltpu.VMEM((_CH,), jnp.int32),
            pltpu.VMEM((n2,), jnp.float32),
            pltpu.VMEM((NS, _CW), jnp.float32),
            pltpu.VMEM((_CW,), jnp.float32),
            pltpu.VMEM((n2,), jnp.float32),
            pltpu.VMEM_SHARED((NS, n2), jnp.float32),
            pltpu.VMEM_SHARED((n2,), jnp.float32),
            pltpu.SemaphoreType.DMA,
            pltpu.SemaphoreType.DMA,
            pltpu.SemaphoreType.DMA,
        ],
        compiler_params=_SC_PARAMS,
    )(gtraw, packed)


def _scatter_body(n, e, gt_hbm, pk_hbm, parts_hbm,
                  hcol, acc, pkbuf0, pkbuf1, sem0, sem1):
    cid = lax.axis_index("c")
    sid = lax.axis_index("s")
    eps = e // NC
    base = cid * eps
    nch = eps // _CH
    pkbufs = (pkbuf0, pkbuf1)
    sems = (sem0, sem1)

    def start(c, slot):
        off = base + lax.rem(c + sid, nch) * _CH
        cp = pltpu.make_async_copy(pk_hbm.at[pl.ds(off, _CH)],
                                   pkbufs[slot], sems[slot])
        cp.start()
        return cp

    pending = [None, None]
    pending[0] = start(0, 0)
    pltpu.sync_copy(gt_hbm.at[sid], hcol)
    zeros = jnp.zeros((L,), jnp.float32)

    @plsc.parallel_loop(0, n // L, unroll=8)
    def _(i):
        acc[pl.ds(i * L, L)] = zeros

    mask = jnp.full((L,), 0xFFFF, jnp.int32)

    for c in range(nch):
        slot = c % 2
        if c + 1 < nch:
            pending[(c + 1) % 2] = start(c + 1, (c + 1) % 2)
        pending[slot].wait()
        pb = pkbufs[slot]

        @plsc.parallel_loop(0, _CH // L, unroll=16)
        def _(i):
            p = pb[pl.ds(i * L, L)]
            s16 = lax.bitwise_and(p, mask)
            d16 = lax.shift_right_logical(p, 16)
            v = plsc.load_gather(hcol, [s16])
            plsc.addupdate_scatter(acc, [d16], v)

    pltpu.sync_copy(acc, parts_hbm.at[cid, sid])


def _scatter_call(gt, packed, n, e):
    body = functools.partial(_scatter_body, n, e)
    return pl.kernel(
        body,
        out_type=jax.ShapeDtypeStruct((NC, NS, n), jnp.float32),
        mesh=_MESH,
        scratch_types=[
            pltpu.VMEM((n,), jnp.float32),
            pltpu.VMEM((n,), jnp.float32),
            pltpu.VMEM((_CH,), jnp.int32),
            pltpu.VMEM((_CH,), jnp.int32),
            pltpu.SemaphoreType.DMA,
            pltpu.SemaphoreType.DMA,
        ],
        compiler_params=_SC_PARAMS,
    )(gt, packed)


_PREC = lax.Precision.HIGHEST


def _pack_mm_body(ei_ref, x_ref, w1_ref, pk_ref, graw_ref):
    pk_ref[...] = ei_ref[0] + (ei_ref[1] << 16)
    graw_ref[...] = lax.dot_general(w1_ref[...], x_ref[...],
                                    (((0,), (1,)), ((), ())),
                                    precision=_PREC,
                                    preferred_element_type=jnp.float32)


def _pack_mm_call(edge_index, x, w1, n, h, e):
    return pl.pallas_call(
        _pack_mm_body,
        out_shape=[
            jax.ShapeDtypeStruct((e,), jnp.int32),
            jax.ShapeDtypeStruct((h, n), jnp.float32),
        ],
    )(edge_index, x, w1)


def _mid_body(parts_ref, graw_ref, dinv_ref, w2_ref, b1_ref, out_ref):
    dinv = dinv_ref[...][:, :graw_ref.shape[1]]
    tot = parts_ref[0] + parts_ref[1] + graw_ref[...] * dinv
    hid = jnp.maximum(tot * dinv + b1_ref[...], 0.0)
    g2 = lax.dot_general(w2_ref[...], hid,
                         (((0,), (0,)), ((), ())),
                         precision=_PREC,
                         preferred_element_type=jnp.float32)
    out_ref[...] = g2 * dinv


def _mid_call(parts, g, dinv, w2, b1, n, h):
    return pl.pallas_call(
        _mid_body,
        out_shape=jax.ShapeDtypeStruct((h, n), jnp.float32),
    )(parts, g, dinv, w2, b1.reshape(h, 1))


def _fin_body(parts_ref, g_ref, dinv_ref, wf_ref, b2_ref, bf_ref, out_ref):
    tot = parts_ref[0] + parts_ref[1] + g_ref[...]
    dinv = dinv_ref[...][:, :g_ref.shape[1]]
    hid = jnp.maximum(tot * dinv + b2_ref[...], 0.0)
    o = lax.dot_general(wf_ref[...], hid,
                        (((0,), (0,)), ((), ())),
                        precision=_PREC,
                        preferred_element_type=jnp.float32)
    out_ref[...] = jax.nn.sigmoid(o + bf_ref[...])


def _fin_call(parts, g, dinv, wf, b2, bf, n, h):
    return pl.pallas_call(
        _fin_body,
        out_shape=jax.ShapeDtypeStruct((1, n), jnp.float32),
    )(parts, g, dinv, wf, b2.reshape(h, 1), bf.reshape(1, 1))


def kernel(x, edge_index, edge_attr, batch, W1, b1, W2, b2, Wf, bf):
    n = x.shape[0]
    e = edge_index.shape[1]
    h = W1.shape[1]
    packed, g1raw = _pack_mm_call(edge_index, x, W1, n, h, e)
    parts1, dinv = _fused_call(g1raw, packed, n, e)
    g2 = _mid_call(parts1, g1raw, dinv, W2, b1, n, h)
    parts2 = _scatter_call(g2, packed, n, e)
    out_t = _fin_call(parts2, g2, dinv, Wf, b2, bf, n, h)
    return out_t.reshape(n, 1)
